# baseline (speedup 1.0000x reference)
import functools

import jax
import jax.numpy as jnp
from jax import lax
from jax.experimental import pallas as pl
from jax.experimental.pallas import tpu as pltpu
from jax.experimental.pallas import tpu_sc as plsc

_GRP = 128
_NW = 32
_B = 4096
_A = 20
_S = 50
_CARD_N = _B * _S
_ACT_N = _B * _A * 8
_ABATCH = 8
_AROWS = _ABATCH * _GRP
_UPS = _B // 16
_AI = (_A * _UPS) // (_NW * _ABATCH)
_CBLK = _B // _GRP
_CPW = (_S * _CBLK) // _NW


def _sc_prepare(table, cids_t, aids_t, nums_t):
    mesh = plsc.VectorSubcoreMesh(core_axis_name="c", subcore_axis_name="s")

    @functools.partial(
        pl.kernel,
        mesh=mesh,
        out_type=[
            jax.ShapeDtypeStruct((_S, 80, _B), jnp.float32),
            jax.ShapeDtypeStruct((_ACT_N, 64), jnp.float32),
        ],
        scratch_types=[
            pltpu.VMEM((_ABATCH, _GRP), jnp.int32),
            pltpu.VMEM((1, 8, _GRP), jnp.int32),
            pltpu.VMEM((_AROWS, 64), jnp.float32),
            pltpu.VMEM((_CPW, _GRP), jnp.int32),
            pltpu.VMEM((2, _GRP, 64), jnp.float32),
            pltpu.VMEM((3, 1, 80, _GRP), jnp.float32),
            pltpu.SemaphoreType.DMA,
            pltpu.SemaphoreType.DMA,
            pltpu.SemaphoreType.DMA,
            pltpu.SemaphoreType.DMA,
        ],
        compiler_params=pltpu.CompilerParams(
            use_tc_tiling_on_sc=False, needs_layout_passes=False),
    )
    def k(table_h, cids_h, aids_h, nums_h, outc_h, outa_h,
          aidx_v, araw_v, arows_v, cidx_v, crows_v, cblk_v,
          asem, gsem, nsem, wsem):
        w = lax.axis_index("s") * 2 + lax.axis_index("c")

        lane = lax.broadcasted_iota(jnp.int32, (16,), 0)
        l_vec = lane % 8
        b_vec = lane // 8

        def abody(i, carry):
            u0 = w * (_AI * _ABATCH) + i * _ABATCH
            a = u0 // _UPS
            b0 = (u0 - a * _UPS) * 16
            pltpu.sync_copy(aids_h.at[pl.ds(a, 1), :, pl.ds(b0, _GRP)],
                            araw_v)
            def tbody(g, c2):
                for c in range(8):
                    vals = plsc.load_gather(
                        araw_v,
                        [jnp.zeros((16,), jnp.int32), l_vec,
                         16 * g + 2 * c + b_vec])
                    aidx_v[g, pl.ds(16 * c, 16)] = vals
                return c2
            lax.fori_loop(0, _ABATCH, tbody, 0)

            hs = []
            for g in range(_ABATCH):
                hs.append(pltpu.async_copy(
                    table_h.at[aidx_v.at[g]],
                    arows_v.at[pl.ds(g * _GRP, _GRP)], asem))
            for h in hs:
                h.wait()
            pltpu.sync_copy(arows_v, outa_h.at[pl.ds(u0 * _GRP, _AROWS)])
            return carry

        lax.fori_loop(0, _AI, abody, 0)

        blk0 = w * _CPW

        def sbody(kk, carry):
            blk = blk0 + kk
            s = blk // _CBLK
            bb = (blk - s * _CBLK) * _GRP
            pltpu.sync_copy(cids_h.at[pl.ds(s, 1), pl.ds(bb, _GRP)],
                            cidx_v.at[pl.ds(kk, 1)])
            return carry

        lax.fori_loop(0, _CPW, sbody, 0)

        def _addr(kk):
            blk = blk0 + kk
            s = blk // _CBLK
            bb = (blk - s * _CBLK) * _GRP
            return s, bb

        def _fire(kk):
            s, bb = _addr(kk)
            pltpu.async_copy(table_h.at[cidx_v.at[kk]],
                             crows_v.at[kk % 2], gsem)
            pltpu.async_copy(
                nums_h.at[pl.ds(s, 1), :, pl.ds(bb, _GRP)],
                cblk_v.at[kk % 3, :, pl.ds(64, 16)], nsem)

        def _wait_write(kk):
            s, bb = _addr(kk)
            pltpu.make_async_copy(
                cblk_v.at[kk % 3],
                outc_h.at[pl.ds(s, 1), :, pl.ds(bb, _GRP)], wsem).wait()

        _fire(0)

        def cbody(kk, carry):
            s, bb = _addr(kk)
            bank = kk % 3

            @pl.when(kk >= 2)
            def _():
                _wait_write(kk - 2)

            @pl.when(kk < _CPW - 1)
            def _():
                _fire(kk + 1)

            pltpu.make_async_copy(table_h.at[cidx_v.at[kk]],
                                  crows_v.at[kk % 2], gsem).wait()
            pltpu.make_async_copy(
                nums_h.at[pl.ds(s, 1), :, pl.ds(bb, _GRP)],
                cblk_v.at[bank, :, pl.ds(64, 16)], nsem).wait()

            def dbody(d, c2):
                for c in range(8):
                    vals = plsc.load_gather(
                        crows_v.at[kk % 2],
                        [lane + 16 * c, jnp.full((16,), 0, jnp.int32) + d])
                    cblk_v[bank, 0, d, pl.ds(16 * c, 16)] = vals
                return c2
            lax.fori_loop(0, 64, dbody, 0)

            pltpu.async_copy(cblk_v.at[bank],
                             outc_h.at[pl.ds(s, 1), :, pl.ds(bb, _GRP)], wsem)
            return carry

        lax.fori_loop(0, _CPW, cbody, 0)

        _wait_write(_CPW - 2)
        _wait_write(_CPW - 1)

    return k(table, cids_t, aids_t, nums_t)


def kernel(reals, card_nums, embed_table, cardIDs, actionIDs):
    cids_t = cardIDs.astype(jnp.int32).transpose(1, 0)
    aids_t = actionIDs.astype(jnp.int32).transpose(1, 2, 0)
    nums_t = card_nums.transpose(1, 2, 0)
    out_c, out_a = _sc_prepare(embed_table, cids_t, aids_t, nums_t)
    card = out_c.transpose(2, 0, 1)
    act = out_a.reshape(_A, _B, 512).transpose(1, 0, 2)
    return (reals, card, act)


# speedup vs baseline: 1.3455x; 1.3455x over previous
import functools

import jax
import jax.numpy as jnp
from jax import lax
from jax.experimental import pallas as pl
from jax.experimental.pallas import tpu as pltpu
from jax.experimental.pallas import tpu_sc as plsc

_GRP = 128
_NW = 32
_B = 4096
_A = 20
_S = 50
_CARD_N = _B * _S
_ACT_N = _B * _A * 8
_ABATCH = 8
_AROWS = _ABATCH * _GRP
_UPS = _B // 16
_AI = (_A * _UPS) // (_NW * _ABATCH)
_CBLK = _B // _GRP
_CPW = (_S * _CBLK) // _NW


def _sc_prepare(table, cids_t, aids_t, nums_t):
    mesh = plsc.VectorSubcoreMesh(core_axis_name="c", subcore_axis_name="s")

    @functools.partial(
        pl.kernel,
        mesh=mesh,
        out_type=[
            jax.ShapeDtypeStruct((_S, 80, _B), jnp.float32),
            jax.ShapeDtypeStruct((_ACT_N, 64), jnp.float32),
        ],
        scratch_types=[
            pltpu.VMEM((_ABATCH, _GRP), jnp.int32),
            pltpu.VMEM((1, 8, _GRP), jnp.int32),
            pltpu.VMEM((_AROWS, 64), jnp.float32),
            pltpu.VMEM((_CPW, _GRP), jnp.int32),
            pltpu.VMEM((2, _GRP, 64), jnp.float32),
            pltpu.VMEM((3, 1, 80, _GRP), jnp.float32),
            pltpu.SemaphoreType.DMA,
            pltpu.SemaphoreType.DMA,
            pltpu.SemaphoreType.DMA,
            pltpu.SemaphoreType.DMA,
        ],
        compiler_params=pltpu.CompilerParams(
            use_tc_tiling_on_sc=False, needs_layout_passes=False),
    )
    def k(table_h, cids_h, aids_h, nums_h, outc_h, outa_h,
          aidx_v, araw_v, arows_v, cidx_v, crows_v, cblk_v,
          asem, gsem, nsem, wsem):
        w = lax.axis_index("s") * 2 + lax.axis_index("c")

        lane = lax.broadcasted_iota(jnp.int32, (16,), 0)
        l_vec = lane % 8
        b_vec = lane // 8

        def abody(i, carry):
            u0 = w * (_AI * _ABATCH) + i * _ABATCH
            a = u0 // _UPS
            b0 = (u0 - a * _UPS) * 16
            pltpu.sync_copy(aids_h.at[pl.ds(a, 1), :, pl.ds(b0, _GRP)],
                            araw_v)
            def tbody(g, c2):
                for c in range(8):
                    vals = plsc.load_gather(
                        araw_v,
                        [jnp.zeros((16,), jnp.int32), l_vec,
                         16 * g + 2 * c + b_vec])
                    aidx_v[g, pl.ds(16 * c, 16)] = vals
                return c2
            lax.fori_loop(0, _ABATCH, tbody, 0)

            hs = []
            for g in range(_ABATCH):
                hs.append(pltpu.async_copy(
                    table_h.at[aidx_v.at[g]],
                    arows_v.at[pl.ds(g * _GRP, _GRP)], asem))
            for h in hs:
                h.wait()
            pltpu.sync_copy(arows_v, outa_h.at[pl.ds(u0 * _GRP, _AROWS)])
            return carry

        lax.fori_loop(0, _AI, abody, 0)

        blk0 = w * _CPW

        def sbody(kk, carry):
            blk = blk0 + kk
            s = blk // _CBLK
            bb = (blk - s * _CBLK) * _GRP
            pltpu.sync_copy(cids_h.at[pl.ds(s, 1), pl.ds(bb, _GRP)],
                            cidx_v.at[pl.ds(kk, 1)])
            return carry

        lax.fori_loop(0, _CPW, sbody, 0)

        def _addr(kk):
            blk = blk0 + kk
            s = blk // _CBLK
            bb = (blk - s * _CBLK) * _GRP
            return s, bb

        def _fire(kk):
            s, bb = _addr(kk)
            pltpu.async_copy(table_h.at[cidx_v.at[kk]],
                             crows_v.at[kk % 2], gsem)
            pltpu.async_copy(
                nums_h.at[pl.ds(s, 1), :, pl.ds(bb, _GRP)],
                cblk_v.at[kk % 3, :, pl.ds(64, 16)], nsem)

        def _wait_write(kk):
            s, bb = _addr(kk)
            pltpu.make_async_copy(
                cblk_v.at[kk % 3],
                outc_h.at[pl.ds(s, 1), :, pl.ds(bb, _GRP)], wsem).wait()

        _fire(0)

        def cbody(kk, carry):
            s, bb = _addr(kk)
            bank = kk % 3

            @pl.when(kk >= 2)
            def _():
                _wait_write(kk - 2)

            @pl.when(kk < _CPW - 1)
            def _():
                _fire(kk + 1)

            pltpu.make_async_copy(table_h.at[cidx_v.at[kk]],
                                  crows_v.at[kk % 2], gsem).wait()
            pltpu.make_async_copy(
                nums_h.at[pl.ds(s, 1), :, pl.ds(bb, _GRP)],
                cblk_v.at[bank, :, pl.ds(64, 16)], nsem).wait()

            src2 = crows_v.at[kk % 2]
            dst2 = cblk_v.at[bank, 0]

            def tile_body(ti, c2):
                j0 = (ti // 4) * 16
                d0 = (ti - (ti // 4) * 4) * 16
                jv = j0 + lane
                for t in range(16):
                    dv = d0 + ((lane + t) & 15)
                    vals = plsc.load_gather(src2, [jv, dv])
                    plsc.store_scatter(dst2, [dv, jv], vals)
                return c2
            lax.fori_loop(0, 32, tile_body, 0)

            pltpu.async_copy(cblk_v.at[bank],
                             outc_h.at[pl.ds(s, 1), :, pl.ds(bb, _GRP)], wsem)
            return carry

        lax.fori_loop(0, _CPW, cbody, 0)

        _wait_write(_CPW - 2)
        _wait_write(_CPW - 1)

    return k(table, cids_t, aids_t, nums_t)


def kernel(reals, card_nums, embed_table, cardIDs, actionIDs):
    cids_t = cardIDs.astype(jnp.int32).transpose(1, 0)
    aids_t = actionIDs.astype(jnp.int32).transpose(1, 2, 0)
    nums_t = card_nums.transpose(1, 2, 0)
    out_c, out_a = _sc_prepare(embed_table, cids_t, aids_t, nums_t)
    card = out_c.transpose(2, 0, 1)
    act = out_a.reshape(_A, _B, 512).transpose(1, 0, 2)
    return (reals, card, act)


# speedup vs baseline: 1.6420x; 1.2203x over previous
import functools

import jax
import jax.numpy as jnp
from jax import lax
from jax.experimental import pallas as pl
from jax.experimental.pallas import tpu as pltpu
from jax.experimental.pallas import tpu_sc as plsc

_GRP = 128
_NW = 32
_B = 4096
_A = 20
_S = 50
_CARD_N = _B * _S
_ACT_N = _B * _A * 8
_ABATCH = 8
_AROWS = _ABATCH * _GRP
_UPS = _B // 16
_AI = (_A * _UPS) // (_NW * _ABATCH)
_CBLK = _B // _GRP
_CPW = (_S * _CBLK) // _NW

_MESH = dict(core_axis_name="c", subcore_axis_name="s")
_PARAMS = pltpu.CompilerParams(
    use_tc_tiling_on_sc=False, needs_layout_passes=False)


def _sc_actions(table, aids_t):
    @functools.partial(
        pl.kernel,
        mesh=plsc.VectorSubcoreMesh(**_MESH),
        out_type=jax.ShapeDtypeStruct((_ACT_N, 64), jnp.float32),
        scratch_types=[
            pltpu.VMEM((_ABATCH, _GRP), jnp.int32),
            pltpu.VMEM((1, 8, _GRP), jnp.int32),
            pltpu.VMEM((_AROWS, 64), jnp.float32),
            pltpu.SemaphoreType.DMA,
        ],
        compiler_params=_PARAMS,
    )
    def k(table_h, aids_h, outa_h, aidx_v, araw_v, arows_v, asem):
        w = lax.axis_index("s") * 2 + lax.axis_index("c")
        lane = lax.broadcasted_iota(jnp.int32, (16,), 0)
        l_vec = lane % 8
        b_vec = lane // 8

        def abody(i, carry):
            u0 = w * (_AI * _ABATCH) + i * _ABATCH
            a = u0 // _UPS
            b0 = (u0 - a * _UPS) * 16
            pltpu.sync_copy(aids_h.at[pl.ds(a, 1), :, pl.ds(b0, _GRP)],
                            araw_v)

            def tbody(g, c2):
                for c in range(8):
                    vals = plsc.load_gather(
                        araw_v,
                        [jnp.zeros((16,), jnp.int32), l_vec,
                         16 * g + 2 * c + b_vec])
                    aidx_v[g, pl.ds(16 * c, 16)] = vals
                return c2
            lax.fori_loop(0, _ABATCH, tbody, 0)

            hs = []
            for g in range(_ABATCH):
                hs.append(pltpu.async_copy(
                    table_h.at[aidx_v.at[g]],
                    arows_v.at[pl.ds(g * _GRP, _GRP)], asem))
            for h in hs:
                h.wait()
            pltpu.sync_copy(arows_v, outa_h.at[pl.ds(u0 * _GRP, _AROWS)])
            return carry

        lax.fori_loop(0, _AI, abody, 0)

    return k(table, aids_t)


def _sc_cards(table, cids_t, nums_t):
    @functools.partial(
        pl.kernel,
        mesh=plsc.VectorSubcoreMesh(**_MESH),
        out_type=jax.ShapeDtypeStruct((_S, 80, _B), jnp.float32),
        scratch_types=[
            pltpu.VMEM((_CPW, _GRP), jnp.int32),
            pltpu.VMEM((2, _GRP, 64), jnp.float32),
            pltpu.VMEM((3, 1, 80, _GRP), jnp.float32),
            pltpu.SemaphoreType.DMA,
            pltpu.SemaphoreType.DMA,
            pltpu.SemaphoreType.DMA,
        ],
        compiler_params=_PARAMS,
    )
    def k(table_h, cids_h, nums_h, outc_h, cidx_v, crows_v, cblk_v,
          gsem, nsem, wsem):
        w = lax.axis_index("s") * 2 + lax.axis_index("c")
        lane = lax.broadcasted_iota(jnp.int32, (16,), 0)
        blk0 = w * _CPW

        def _addr(kk):
            blk = blk0 + kk
            s = blk // _CBLK
            bb = (blk - s * _CBLK) * _GRP
            return s, bb

        def sbody(kk, carry):
            s, bb = _addr(kk)
            pltpu.sync_copy(cids_h.at[pl.ds(s, 1), pl.ds(bb, _GRP)],
                            cidx_v.at[pl.ds(kk, 1)])
            return carry

        lax.fori_loop(0, _CPW, sbody, 0)

        def _fire(kk):
            s, bb = _addr(kk)
            pltpu.async_copy(table_h.at[cidx_v.at[kk]],
                             crows_v.at[kk % 2], gsem)
            pltpu.async_copy(
                nums_h.at[pl.ds(s, 1), :, pl.ds(bb, _GRP)],
                cblk_v.at[kk % 3, :, pl.ds(64, 16)], nsem)

        def _wait_write(kk):
            s, bb = _addr(kk)
            pltpu.make_async_copy(
                cblk_v.at[kk % 3],
                outc_h.at[pl.ds(s, 1), :, pl.ds(bb, _GRP)], wsem).wait()

        _fire(0)

        def cbody(kk, carry):
            s, bb = _addr(kk)
            bank = kk % 3

            @pl.when(kk >= 2)
            def _():
                _wait_write(kk - 2)

            @pl.when(kk < _CPW - 1)
            def _():
                _fire(kk + 1)

            pltpu.make_async_copy(table_h.at[cidx_v.at[kk]],
                                  crows_v.at[kk % 2], gsem).wait()
            pltpu.make_async_copy(
                nums_h.at[pl.ds(s, 1), :, pl.ds(bb, _GRP)],
                cblk_v.at[bank, :, pl.ds(64, 16)], nsem).wait()

            src2 = crows_v.at[kk % 2]
            dst2 = cblk_v.at[bank, 0]

            def tile_body(ti, c2):
                j0 = (ti // 4) * 16
                d0 = (ti - (ti // 4) * 4) * 16
                jv = j0 + lane
                for t in range(16):
                    dv = d0 + ((lane + t) & 15)
                    vals = plsc.load_gather(src2, [jv, dv])
                    plsc.store_scatter(dst2, [dv, jv], vals)
                return c2
            lax.fori_loop(0, 32, tile_body, 0)

            pltpu.async_copy(cblk_v.at[bank],
                             outc_h.at[pl.ds(s, 1), :, pl.ds(bb, _GRP)], wsem)
            return carry

        lax.fori_loop(0, _CPW, cbody, 0)

        _wait_write(_CPW - 2)
        _wait_write(_CPW - 1)

    return k(table, cids_t, nums_t)


def kernel(reals, card_nums, embed_table, cardIDs, actionIDs):
    cids_t = cardIDs.astype(jnp.int32).transpose(1, 0)
    aids_t = actionIDs.astype(jnp.int32).transpose(1, 2, 0)
    nums_t = card_nums.transpose(1, 2, 0)
    out_a = _sc_actions(embed_table, aids_t)
    out_c = _sc_cards(embed_table, cids_t, nums_t)
    card = out_c.transpose(2, 0, 1)
    act = out_a.reshape(_A, _B, 512).transpose(1, 0, 2)
    return (reals, card, act)


# speedup vs baseline: 1.7392x; 1.0592x over previous
import functools

import jax
import jax.numpy as jnp
from jax import lax
from jax.experimental import pallas as pl
from jax.experimental.pallas import tpu as pltpu
from jax.experimental.pallas import tpu_sc as plsc

_GRP = 128
_NW = 32
_B = 4096
_A = 20
_S = 50
_CARD_N = _B * _S
_ACT_N = _B * _A * 8
_ABATCH = 4
_AROWS = _ABATCH * _GRP
_UPS = _B // 16
_AI = (_A * _UPS) // (_NW * _ABATCH)
_CBLK = _B // _GRP
_CPW = (_S * _CBLK) // _NW

_MESH = dict(core_axis_name="c", subcore_axis_name="s")
_PARAMS = pltpu.CompilerParams(
    use_tc_tiling_on_sc=False, needs_layout_passes=False)


def _sc_actions(table, aids_t):
    @functools.partial(
        pl.kernel,
        mesh=plsc.VectorSubcoreMesh(**_MESH),
        out_type=jax.ShapeDtypeStruct((_ACT_N, 64), jnp.float32),
        scratch_types=[
            pltpu.VMEM((2, _ABATCH, _GRP), jnp.int32),
            pltpu.VMEM((1, 8, _ABATCH * 16), jnp.int32),
            pltpu.VMEM((2, _AROWS, 64), jnp.float32),
            pltpu.SemaphoreType.DMA,
            pltpu.SemaphoreType.DMA,
        ],
        compiler_params=_PARAMS,
    )
    def k(table_h, aids_h, outa_h, aidx_v, araw_v, arows_v, asem, wsem):
        w = lax.axis_index("s") * 2 + lax.axis_index("c")
        lane = lax.broadcasted_iota(jnp.int32, (16,), 0)
        l_vec = lane % 8
        b_vec = lane // 8
        u_base = w * (_AI * _ABATCH)

        def _stage(i):
            u0 = u_base + i * _ABATCH
            a = u0 // _UPS
            b0 = (u0 - a * _UPS) * 16
            pltpu.sync_copy(
                aids_h.at[pl.ds(a, 1), :, pl.ds(b0, _ABATCH * 16)], araw_v)

            def tbody(g, c2):
                for c in range(8):
                    vals = plsc.load_gather(
                        araw_v,
                        [jnp.zeros((16,), jnp.int32), l_vec,
                         16 * g + 2 * c + b_vec])
                    aidx_v[i % 2, g, pl.ds(16 * c, 16)] = vals
                return c2
            lax.fori_loop(0, _ABATCH, tbody, 0)

        def _fire(i):
            for g in range(_ABATCH):
                pltpu.async_copy(
                    table_h.at[aidx_v.at[i % 2, g]],
                    arows_v.at[i % 2, pl.ds(g * _GRP, _GRP)], asem)

        def _wait_gathers(i):
            for g in range(_ABATCH):
                pltpu.make_async_copy(
                    table_h.at[aidx_v.at[i % 2, g]],
                    arows_v.at[i % 2, pl.ds(g * _GRP, _GRP)], asem).wait()

        def _out_slice(i):
            return outa_h.at[pl.ds((u_base + i * _ABATCH) * _GRP, _AROWS)]

        _stage(0)
        _fire(0)

        def abody(i, carry):
            @pl.when(i < _AI - 1)
            def _():
                _stage(i + 1)

            @pl.when(i >= 1)
            def _():
                pltpu.make_async_copy(arows_v.at[(i - 1) % 2],
                                      _out_slice(i - 1), wsem).wait()

            @pl.when(i < _AI - 1)
            def _():
                _fire(i + 1)

            _wait_gathers(i)
            pltpu.async_copy(arows_v.at[i % 2], _out_slice(i), wsem)
            return carry

        lax.fori_loop(0, _AI, abody, 0)
        pltpu.make_async_copy(arows_v.at[(_AI - 1) % 2],
                              _out_slice(_AI - 1), wsem).wait()

    return k(table, aids_t)


def _sc_cards(table, cids_t, nums_t):
    @functools.partial(
        pl.kernel,
        mesh=plsc.VectorSubcoreMesh(**_MESH),
        out_type=jax.ShapeDtypeStruct((_S, 80, _B), jnp.float32),
        scratch_types=[
            pltpu.VMEM((_CPW, _GRP), jnp.int32),
            pltpu.VMEM((2, _GRP, 64), jnp.float32),
            pltpu.VMEM((3, 1, 80, _GRP), jnp.float32),
            pltpu.SemaphoreType.DMA,
            pltpu.SemaphoreType.DMA,
            pltpu.SemaphoreType.DMA,
        ],
        compiler_params=_PARAMS,
    )
    def k(table_h, cids_h, nums_h, outc_h, cidx_v, crows_v, cblk_v,
          gsem, nsem, wsem):
        w = lax.axis_index("s") * 2 + lax.axis_index("c")
        lane = lax.broadcasted_iota(jnp.int32, (16,), 0)
        blk0 = w * _CPW

        def _addr(kk):
            blk = blk0 + kk
            s = blk // _CBLK
            bb = (blk - s * _CBLK) * _GRP
            return s, bb

        def sbody(kk, carry):
            s, bb = _addr(kk)
            pltpu.sync_copy(cids_h.at[pl.ds(s, 1), pl.ds(bb, _GRP)],
                            cidx_v.at[pl.ds(kk, 1)])
            return carry

        lax.fori_loop(0, _CPW, sbody, 0)

        def _fire(kk):
            s, bb = _addr(kk)
            pltpu.async_copy(table_h.at[cidx_v.at[kk]],
                             crows_v.at[kk % 2], gsem)
            pltpu.async_copy(
                nums_h.at[pl.ds(s, 1), :, pl.ds(bb, _GRP)],
                cblk_v.at[kk % 3, :, pl.ds(64, 16)], nsem)

        def _wait_write(kk):
            s, bb = _addr(kk)
            pltpu.make_async_copy(
                cblk_v.at[kk % 3],
                outc_h.at[pl.ds(s, 1), :, pl.ds(bb, _GRP)], wsem).wait()

        _fire(0)

        def cbody(kk, carry):
            s, bb = _addr(kk)
            bank = kk % 3

            @pl.when(kk >= 2)
            def _():
                _wait_write(kk - 2)

            @pl.when(kk < _CPW - 1)
            def _():
                _fire(kk + 1)

            pltpu.make_async_copy(table_h.at[cidx_v.at[kk]],
                                  crows_v.at[kk % 2], gsem).wait()
            pltpu.make_async_copy(
                nums_h.at[pl.ds(s, 1), :, pl.ds(bb, _GRP)],
                cblk_v.at[bank, :, pl.ds(64, 16)], nsem).wait()

            src2 = crows_v.at[kk % 2]
            dst2 = cblk_v.at[bank, 0]

            def tile_body(ti, c2):
                j0 = (ti // 4) * 16
                d0 = (ti - (ti // 4) * 4) * 16
                jv = j0 + lane
                for t in range(16):
                    dv = d0 + ((lane + t) & 15)
                    vals = plsc.load_gather(src2, [jv, dv])
                    plsc.store_scatter(dst2, [dv, jv], vals)
                return c2
            lax.fori_loop(0, 32, tile_body, 0)

            pltpu.async_copy(cblk_v.at[bank],
                             outc_h.at[pl.ds(s, 1), :, pl.ds(bb, _GRP)], wsem)
            return carry

        lax.fori_loop(0, _CPW, cbody, 0)

        _wait_write(_CPW - 2)
        _wait_write(_CPW - 1)

    return k(table, cids_t, nums_t)


def kernel(reals, card_nums, embed_table, cardIDs, actionIDs):
    cids_t = cardIDs.astype(jnp.int32).transpose(1, 0)
    aids_t = actionIDs.astype(jnp.int32).transpose(1, 2, 0)
    nums_t = card_nums.transpose(1, 2, 0)
    out_a = _sc_actions(embed_table, aids_t)
    out_c = _sc_cards(embed_table, cids_t, nums_t)
    card = out_c.transpose(2, 0, 1)
    act = out_a.reshape(_A, _B, 512).transpose(1, 0, 2)
    return (reals, card, act)
